# baseline (speedup 1.0000x reference)
import jax
import jax.numpy as jnp
from jax.experimental import pallas as pl
from jax.experimental.pallas import tpu as pltpu

C = 64
BN = 18432
NC = 6


def _make_body(nb):
    def _fused_kernel(gamma_ref, x_ref, prev_ref, o_ref, xsave, g_s, aff_s):
        del prev_ref
        b = pl.program_id(0)
        i = pl.program_id(1)

        xs = xsave[i]
        w = jnp.dot(aff_s[...], xs, preferred_element_type=jnp.float32)
        o_ref[0] = gamma_ref[0] * w + xs

        @pl.when(b < nb)
        def _():
            xb = x_ref[0]

            @pl.when(i == 0)
            def _():
                g_s[...] = jnp.zeros_like(g_s)

            g_s[...] += jax.lax.dot_general(
                xb, xb, (((1,), (1,)), ((), ())),
                preferred_element_type=jnp.float32)

            @pl.when(i == NC - 1)
            def _():
                g = g_s[...]
                m3 = jnp.dot(g, g, preferred_element_type=jnp.float32)
                aff_s[...] = jax.nn.sigmoid(m3)

            xsave[i] = xb

    return _fused_kernel


def _lag_call(k, gamma1, prev, b0, nb):
    B = k.shape[0]
    N = k.shape[2]
    return pl.pallas_call(
        _make_body(nb),
        grid=(nb + 1, NC),
        in_specs=[
            pl.BlockSpec(memory_space=pltpu.SMEM),
            pl.BlockSpec(
                (1, C, BN),
                lambda b, i: (b0 + jnp.minimum(b, nb - 1), 0,
                              jnp.where(b < nb, i, NC - 1))),
            pl.BlockSpec(memory_space=pl.ANY),
        ],
        out_specs=pl.BlockSpec(
            (1, C, BN),
            lambda b, i: (b0 + jnp.maximum(b - 1, 0), 0,
                          jnp.where(b >= 1, i, 0))),
        out_shape=jax.ShapeDtypeStruct((B, C, N), jnp.float32),
        scratch_shapes=[
            pltpu.VMEM((NC, C, BN), jnp.float32),
            pltpu.VMEM((C, C), jnp.float32),
            pltpu.VMEM((C, C), jnp.float32),
        ],
        input_output_aliases={2: 0},
        compiler_params=pltpu.CompilerParams(
            dimension_semantics=("arbitrary", "arbitrary"),
            vmem_limit_bytes=50 * 1024 * 1024),
    )(gamma1, k, prev)


def kernel(x, gamma):
    B, W, D, H, Cx = x.shape
    N = W * D * H
    k = x.reshape(B, Cx, N)
    g1 = gamma.reshape(1)
    h = B // 2

    out = jnp.empty((B, C, N), jnp.float32)
    out = _lag_call(k, g1, out, 0, h)
    out = _lag_call(k, g1, out, h, B - h)
    return out.reshape(B, W, D, H, Cx)


# speedup vs baseline: 1.0804x; 1.0804x over previous
import jax
import jax.numpy as jnp
from jax.experimental import pallas as pl
from jax.experimental.pallas import tpu as pltpu

C = 64
BN = 27648
NC = 4


def _fused_kernel(gamma_ref, x_ref, o_ref, xsave, g_s, aff_s):
    nb = pl.num_programs(0) - 1
    b = pl.program_id(0)
    i = pl.program_id(1)

    xs = xsave[i]
    w = jnp.dot(aff_s[...], xs, preferred_element_type=jnp.float32)
    o_ref[0] = gamma_ref[0] * w + xs

    @pl.when(b < nb)
    def _():
        xb = x_ref[0]

        @pl.when(i == 0)
        def _():
            g_s[...] = jnp.zeros_like(g_s)

        g_s[...] += jax.lax.dot_general(
            xb, xb, (((1,), (1,)), ((), ())),
            preferred_element_type=jnp.float32)

        @pl.when(i == NC - 1)
        def _():
            g = g_s[...]
            m3 = jnp.dot(g, g, preferred_element_type=jnp.float32)
            aff_s[...] = jax.nn.sigmoid(m3)

        xsave[i] = xb


def kernel(x, gamma):
    B, W, D, H, Cx = x.shape
    N = W * D * H
    k = x.reshape(B, Cx, N)

    out = pl.pallas_call(
        _fused_kernel,
        grid=(B + 1, NC),
        in_specs=[
            pl.BlockSpec(memory_space=pltpu.SMEM),
            pl.BlockSpec(
                (1, C, BN),
                lambda b, i: (jnp.minimum(b, B - 1), 0,
                              jnp.where(b < B, i, NC - 1))),
        ],
        out_specs=pl.BlockSpec(
            (1, C, BN),
            lambda b, i: (jnp.maximum(b - 1, 0), 0,
                          jnp.where(b >= 1, i, 0))),
        out_shape=jax.ShapeDtypeStruct((B, C, N), jnp.float32),
        scratch_shapes=[
            pltpu.VMEM((NC, C, BN), jnp.float32),
            pltpu.VMEM((C, C), jnp.float32),
            pltpu.VMEM((C, C), jnp.float32),
        ],
        compiler_params=pltpu.CompilerParams(
            dimension_semantics=("arbitrary", "arbitrary"),
            vmem_limit_bytes=63 * 1024 * 1024),
    )(gamma.reshape(1), k)

    return out.reshape(B, W, D, H, Cx)
